# baseline (speedup 1.0000x reference)
import functools

import jax
import jax.numpy as jnp
from jax import lax
from jax.experimental import pallas as pl
from jax.experimental.pallas import tpu as pltpu
from jax.experimental.pallas import tpu_sc as plsc

_INFO = plsc.get_sparse_core_info()
_NC = _INFO.num_cores
_NS = _INFO.num_subcores
_NW = _NC * _NS
_K = 4
_NBUF = 4
_WIDE = 128


def _widen_table(token):
    v, d = token.shape
    rows = 1000
    assert v % rows == 0

    def body(tok_ref, wide_ref):
        wide_ref[:, :d] = tok_ref[...]

    return pl.pallas_call(
        body,
        grid=(v // rows,),
        in_specs=[pl.BlockSpec((rows, d), lambda i: (i, 0))],
        out_specs=pl.BlockSpec((rows, _WIDE), lambda i: (i, 0)),
        out_shape=jax.ShapeDtypeStruct((v, _WIDE), jnp.float32),
    )(token)


def _narrow_out(wide, d):
    b0, b1, _ = wide.shape
    blk = 128
    assert b0 % blk == 0

    def body(wide_ref, out_ref):
        out_ref[...] = wide_ref[:, :, :d]

    return pl.pallas_call(
        body,
        grid=(b0 // blk,),
        in_specs=[pl.BlockSpec((blk, b1, _WIDE), lambda i: (i, 0, 0))],
        out_specs=pl.BlockSpec((blk, b1, d), lambda i: (i, 0, 0)),
        out_shape=jax.ShapeDtypeStruct((b0, b1, d), jnp.float32),
    )(wide)


def _make_gather(num_rows: int, d: int, b0: int, b1: int):
    assert b0 % _NW == 0
    rows_per_w = b0 // _NW
    assert rows_per_w % (_K * _NBUF) == 0
    n_groups = rows_per_w // _K
    n_outer = n_groups // _NBUF
    mesh = plsc.VectorSubcoreMesh(core_axis_name="c", subcore_axis_name="s")

    @functools.partial(
        pl.kernel,
        mesh=mesh,
        out_type=jax.ShapeDtypeStruct((b0, b1, _WIDE), jnp.float32),
        scratch_types=[
            pltpu.VMEM((rows_per_w, b1), jnp.int32),
            pltpu.VMEM((_NBUF, _K, b1, _WIDE), jnp.float32),
            pltpu.SemaphoreType.DMA((_NBUF,)),
            pltpu.SemaphoreType.DMA((_NBUF,)),
        ],
        compiler_params=pltpu.CompilerParams(use_tc_tiling_on_sc=True),
    )
    def gather_kernel(token_hbm, idx_hbm, out_hbm, idx_v, rows_v, sems, wsems):
        wid = lax.axis_index("s") * _NC + lax.axis_index("c")
        base = wid * rows_per_w
        pltpu.sync_copy(idx_hbm.at[pl.ds(base, rows_per_w)], idx_v)

        def start_group(j, b):
            for q in range(_K):
                pltpu.async_copy(
                    token_hbm.at[idx_v.at[j * _K + q]], rows_v.at[b, q], sems.at[b]
                )

        def wait_group(j, b):
            for q in range(_K):
                pltpu.make_async_copy(
                    token_hbm.at[idx_v.at[j * _K + q]], rows_v.at[b, q], sems.at[b]
                ).wait()

        def start_write(j, b):
            pltpu.async_copy(
                rows_v.at[b], out_hbm.at[pl.ds(base + j * _K, _K)], wsems.at[b]
            )

        def wait_write(j, b):
            pltpu.make_async_copy(
                rows_v.at[b], out_hbm.at[pl.ds(base + j * _K, _K)], wsems.at[b]
            ).wait()

        start_group(0, 0)
        start_group(1, 1)

        def outer(g, carry):
            for b in range(_NBUF):
                j = g * _NBUF + b
                wait_group(j, b)
                start_write(j, b)
                bn = (b + 2) % _NBUF

                @pl.when(j >= 2)
                def _():
                    wait_write(j - 2, bn)

                @pl.when(j + 2 < n_groups)
                def _():
                    start_group(j + 2, bn)

            return carry

        lax.fori_loop(0, n_outer, outer, 0)
        wait_write(n_groups - 2, (n_groups - 2) % _NBUF)
        wait_write(n_groups - 1, (n_groups - 1) % _NBUF)

    return gather_kernel


def kernel(x, token):
    b0, b1 = x.shape
    num_rows, d = token.shape
    token_wide = _widen_table(token)
    wide = _make_gather(num_rows, d, b0, b1)(token_wide, x.astype(jnp.int32))
    return _narrow_out(wide, d)


# speedup vs baseline: 1.0713x; 1.0713x over previous
import functools

import jax
import jax.numpy as jnp
from jax import lax
from jax.experimental import pallas as pl
from jax.experimental.pallas import tpu as pltpu
from jax.experimental.pallas import tpu_sc as plsc

_INFO = plsc.get_sparse_core_info()
_NC = _INFO.num_cores
_NS = _INFO.num_subcores
_NW = _NC * _NS
_K = 4
_NBUF = 4
_WIDE = 128


def _widen_table(token):
    v, d = token.shape
    rows = 2000
    assert v % rows == 0

    def body(tok_ref, wide_ref):
        wide_ref[:, :d] = tok_ref[...]

    return pl.pallas_call(
        body,
        grid=(v // rows,),
        in_specs=[pl.BlockSpec((rows, d), lambda i: (i, 0))],
        out_specs=pl.BlockSpec((rows, _WIDE), lambda i: (i, 0)),
        out_shape=jax.ShapeDtypeStruct((v, _WIDE), jnp.float32),
    )(token)


def _narrow_out(wide, d):
    b0, b1, _ = wide.shape
    blk = 256
    assert b0 % blk == 0

    def body(wide_ref, out_ref):
        out_ref[...] = wide_ref[:, :, :d]

    return pl.pallas_call(
        body,
        grid=(b0 // blk,),
        in_specs=[pl.BlockSpec((blk, b1, _WIDE), lambda i: (i, 0, 0))],
        out_specs=pl.BlockSpec((blk, b1, d), lambda i: (i, 0, 0)),
        out_shape=jax.ShapeDtypeStruct((b0, b1, d), jnp.float32),
    )(wide)


def _make_gather(num_rows: int, d: int, b0: int, b1: int):
    assert b0 % _NW == 0
    rows_per_w = b0 // _NW
    assert rows_per_w % (_K * _NBUF) == 0
    n_groups = rows_per_w // _K
    n_outer = n_groups // _NBUF
    mesh = plsc.VectorSubcoreMesh(core_axis_name="c", subcore_axis_name="s")

    @functools.partial(
        pl.kernel,
        mesh=mesh,
        out_type=jax.ShapeDtypeStruct((b0, b1, _WIDE), jnp.float32),
        scratch_types=[
            pltpu.VMEM((rows_per_w, b1), jnp.int32),
            pltpu.VMEM((_NBUF, _K, b1, _WIDE), jnp.float32),
            pltpu.SemaphoreType.DMA((_NBUF,)),
            pltpu.SemaphoreType.DMA((_NBUF,)),
        ],
        compiler_params=pltpu.CompilerParams(use_tc_tiling_on_sc=True),
    )
    def gather_kernel(token_hbm, idx_hbm, out_hbm, idx_v, rows_v, sems, wsems):
        wid = lax.axis_index("s") * _NC + lax.axis_index("c")
        base = wid * rows_per_w
        pltpu.sync_copy(idx_hbm.at[pl.ds(base, rows_per_w)], idx_v)

        def start_group(j, b):
            for q in range(_K):
                pltpu.async_copy(
                    token_hbm.at[idx_v.at[j * _K + q]], rows_v.at[b, q], sems.at[b]
                )

        def wait_group(j, b):
            for q in range(_K):
                pltpu.make_async_copy(
                    token_hbm.at[idx_v.at[j * _K + q]], rows_v.at[b, q], sems.at[b]
                ).wait()

        def start_write(j, b):
            pltpu.async_copy(
                rows_v.at[b], out_hbm.at[pl.ds(base + j * _K, _K)], wsems.at[b]
            )

        def wait_write(j, b):
            pltpu.make_async_copy(
                rows_v.at[b], out_hbm.at[pl.ds(base + j * _K, _K)], wsems.at[b]
            ).wait()

        start_group(0, 0)
        start_group(1, 1)

        def outer(g, carry):
            for b in range(_NBUF):
                j = g * _NBUF + b
                wait_group(j, b)
                start_write(j, b)
                bn = (b + 2) % _NBUF

                @pl.when(j >= 2)
                def _():
                    wait_write(j - 2, bn)

                @pl.when(j + 2 < n_groups)
                def _():
                    start_group(j + 2, bn)

            return carry

        lax.fori_loop(0, n_outer, outer, 0)
        wait_write(n_groups - 2, (n_groups - 2) % _NBUF)
        wait_write(n_groups - 1, (n_groups - 1) % _NBUF)

    return gather_kernel


def kernel(x, token):
    b0, b1 = x.shape
    num_rows, d = token.shape
    token_wide = _widen_table(token)
    wide = _make_gather(num_rows, d, b0, b1)(token_wide, x.astype(jnp.int32))
    return _narrow_out(wide, d)


# speedup vs baseline: 1.8303x; 1.7085x over previous
import functools

import jax
import jax.numpy as jnp
from jax import lax
from jax.experimental import pallas as pl
from jax.experimental.pallas import tpu as pltpu
from jax.experimental.pallas import tpu_sc as plsc

_INFO = plsc.get_sparse_core_info()
_NC = _INFO.num_cores
_NS = _INFO.num_subcores
_NW = _NC * _NS
_K = 8
_NBUF = 2
_WIDE = 128


def _make_gather(num_rows: int, d: int, b0: int, b1: int):
    assert b0 % _NW == 0
    rows_per_w = b0 // _NW
    assert rows_per_w % (_K * _NBUF) == 0
    n_groups = rows_per_w // _K
    n_outer = n_groups // _NBUF
    mesh = plsc.VectorSubcoreMesh(core_axis_name="c", subcore_axis_name="s")

    @functools.partial(
        pl.kernel,
        mesh=mesh,
        out_type=jax.ShapeDtypeStruct((b0, b1, _WIDE), jnp.float32),
        scratch_types=[
            pltpu.VMEM((rows_per_w, b1), jnp.int32),
            pltpu.VMEM((_NBUF, _K, b1, _WIDE), jnp.float32),
            pltpu.SemaphoreType.DMA((_NBUF,)),
        ],
        compiler_params=pltpu.CompilerParams(use_tc_tiling_on_sc=True),
    )
    def gather_kernel(token_hbm, idx_hbm, out_hbm, idx_v, rows_v, sems):
        wid = lax.axis_index("s") * _NC + lax.axis_index("c")
        base = wid * rows_per_w
        pltpu.sync_copy(idx_hbm.at[pl.ds(base, rows_per_w)], idx_v)

        def start_group(j, b):
            for q in range(_K):
                pltpu.async_copy(
                    token_hbm.at[idx_v.at[j * _K + q]], rows_v.at[b, q], sems.at[b]
                )

        def wait_group(j, b):
            for q in range(_K):
                pltpu.make_async_copy(
                    token_hbm.at[idx_v.at[j * _K + q]], rows_v.at[b, q], sems.at[b]
                ).wait()

        for b in range(_NBUF):
            start_group(b, b)

        def outer(g, carry):
            for b in range(_NBUF):
                j = g * _NBUF + b
                wait_group(j, b)
                pltpu.sync_copy(
                    rows_v.at[b], out_hbm.at[pl.ds(base + j * _K, _K)]
                )

                @pl.when(g < n_outer - 1)
                def _():
                    start_group(j + _NBUF, b)

            return carry

        lax.fori_loop(0, n_outer, outer, 0)

    return gather_kernel


def kernel(x, token):
    b0, b1 = x.shape
    num_rows, d = token.shape
    token_wide = jnp.pad(token, ((0, 0), (0, _WIDE - d)))
    wide = _make_gather(num_rows, d, b0, b1)(token_wide, x.astype(jnp.int32))
    return wide[:, :, :d]
